# baseline (speedup 1.0000x reference)
import jax
import jax.numpy as jnp
from jax import lax
from jax.experimental import pallas as pl
from jax.experimental.pallas import tpu as pltpu
from jax.experimental.pallas import tpu_sc as plsc

_C, _H, _W = 9, 180, 240
_WH = _W * _H
_HP = 128
_EV_CHUNK = 8192

_EC = 5000
_ZB = 6480


def _tc_body(t_ref, w1_ref, b1s_ref, w2_ref, b2_ref, w3_ref, b3_ref,
             *o_refs):
    t = t_ref[...][None, :]
    w1 = w1_ref[...]
    w2 = w2_ref[...]
    b2 = b2_ref[...]
    w3 = w3_ref[...]
    b3 = b3_ref[0, 0]
    g = w1 * t
    g2 = jnp.concatenate([g, g], axis=0)
    for j in range((_C + 1) // 2):
        h1 = g2 + b1s_ref[:, j:j + 1]
        h1 = jnp.maximum(h1, 0.1 * h1)
        h2 = jnp.dot(w2, h1, preferred_element_type=jnp.float32) + b2
        h2 = jnp.maximum(h2, 0.1 * h2)
        oa = jnp.sum(h2[:_HP] * w3, axis=0, keepdims=True) + b3
        o_refs[2 * j][...] = (t * oa)[0]
        if 2 * j + 1 < _C:
            ob = jnp.sum(h2[_HP:] * w3, axis=0, keepdims=True) + b3
            o_refs[2 * j + 1][...] = (t * ob)[0]


def _tc_values(t1d, w1p, b1p, w2p, b2p, w3p, b3p):
    nev = t1d.shape[0]
    chunk = _EV_CHUNK
    nblk = pl.cdiv(nev, chunk)
    return pl.pallas_call(
        _tc_body,
        grid=(nblk,),
        in_specs=[
            pl.BlockSpec((chunk,), lambda i: (i,)),
            pl.BlockSpec((_HP, 1), lambda i: (0, 0)),
            pl.BlockSpec((2 * _HP, (_C + 1) // 2), lambda i: (0, 0)),
            pl.BlockSpec((2 * _HP, 2 * _HP), lambda i: (0, 0)),
            pl.BlockSpec((2 * _HP, 1), lambda i: (0, 0)),
            pl.BlockSpec((_HP, 1), lambda i: (0, 0)),
            pl.BlockSpec((1, 1), lambda i: (0, 0)),
        ],
        out_specs=[pl.BlockSpec((chunk,), lambda i: (i,))
                   for _ in range(_C)],
        out_shape=[jax.ShapeDtypeStruct((nev,), jnp.float32)
                   for _ in range(_C)],
    )(t1d, w1p, b1p, w2p, b2p, w3p, b3p)


def _sc_scatter(x1d, y1d, p1d, vals, num_voxels, half):
    nev = x1d.shape[0]
    ev_per_core = nev // 2
    ev_per_tile = ev_per_core // 16
    n_chunks = ev_per_tile // _EC
    nfull = _EC // 16
    has_tail = (_EC % 16) != 0
    gwords = half // 16
    mesh = plsc.VectorSubcoreMesh(core_axis_name="c", subcore_axis_name="s")

    def body(x_hbm, y_hbm, p_hbm, *rest):
        v_hbms = rest[:_C]
        out_hbm = rest[_C]
        (x_v, y_v, p_v, idx_v, val_v, zb_v, grid_s) = rest[_C + 1:]
        cid = lax.axis_index("c")
        sid = lax.axis_index("s")

        def zf(i, carry):
            zb_v[pl.ds(i * 16, 16)] = jnp.zeros((16,), jnp.float32)
            return carry
        lax.fori_loop(0, _ZB // 16, zf, 0)
        gbase = sid * gwords

        def zc(j, carry):
            pltpu.sync_copy(zb_v, grid_s.at[pl.ds(gbase + j * _ZB, _ZB)])
            return carry
        lax.fori_loop(0, gwords // _ZB, zc, 0)
        plsc.subcore_barrier()

        ebase = cid * ev_per_core + sid * ev_per_tile
        batch = cid * 2 + sid // 8
        bterm = batch.astype(jnp.float32) * 777600.0
        loc = cid * half

        def base_of(s):
            xx = x_v[pl.ds(s, 16)]
            yy = y_v[pl.ds(s, 16)]
            pp = p_v[pl.ds(s, 16)]
            p2 = (pp + 1.0) * 0.5
            b = xx + 240.0 * yy
            b = b + 388800.0 * p2
            return b + bterm

        def to_idx(f, off):
            f = jnp.clip(f + off, 0.0, float(num_voxels - 1))
            return f.astype(jnp.int32) - loc

        def sc_chunk(r, carry):
            e0 = ebase + r * _EC
            pltpu.sync_copy(x_hbm.at[pl.ds(e0, _EC)], x_v)
            pltpu.sync_copy(y_hbm.at[pl.ds(e0, _EC)], y_v)
            pltpu.sync_copy(p_hbm.at[pl.ds(e0, _EC)], p_v)

            if has_tail:
                zb_v[pl.ds(0, 16)] = base_of(_EC - 16)

            def bvec(v, carry2):
                s = v * 16
                x_v[pl.ds(s, 16)] = base_of(s)
                return carry2
            lax.fori_loop(0, nfull, bvec, 0)

            for i in range(_C):
                off = float(_WH * i)

                def ivec(v, carry3):
                    s = v * 16
                    idx_v[pl.ds(s, 16)] = to_idx(x_v[pl.ds(s, 16)], off)
                    return carry3
                lax.fori_loop(0, nfull, ivec, 0)
                if has_tail:
                    idx_v[pl.ds(_EC - 16, 16)] = to_idx(zb_v[pl.ds(0, 16)],
                                                        off)
                pltpu.sync_copy(v_hbms[i].at[pl.ds(e0, _EC)], val_v)
                pltpu.sync_copy(val_v, grid_s.at[idx_v], add=True)
            return carry
        lax.fori_loop(0, n_chunks, sc_chunk, 0)
        plsc.subcore_barrier()

        def oc(j, carry):
            pltpu.sync_copy(grid_s.at[pl.ds(gbase + j * _ZB, _ZB)], zb_v)
            pltpu.sync_copy(zb_v,
                            out_hbm.at[pl.ds(cid * half + gbase + j * _ZB,
                                             _ZB)])
            return carry
        lax.fori_loop(0, gwords // _ZB, oc, 0)

    f = pl.kernel(
        body,
        out_type=jax.ShapeDtypeStruct((num_voxels,), jnp.float32),
        mesh=mesh,
        scratch_types=[
            pltpu.VMEM((_EC,), jnp.float32),
            pltpu.VMEM((_EC,), jnp.float32),
            pltpu.VMEM((_EC,), jnp.float32),
            pltpu.VMEM((_EC,), jnp.int32),
            pltpu.VMEM((_EC,), jnp.float32),
            pltpu.VMEM((_ZB,), jnp.float32),
            pltpu.VMEM_SHARED((half,), jnp.float32),
        ],
    )
    return f(x1d, y1d, p1d, *vals)


def kernel(events_list, W1, b1, W2, b2, W3, b3):
    Bn, Nn = events_list.shape[0], events_list.shape[1]
    num_voxels = 2 * _C * _H * _W * Bn
    half = num_voxels // 2

    x1d = events_list[:, :, 0].reshape(-1)
    y1d = events_list[:, :, 1].reshape(-1)
    t1d = events_list[:, :, 2].reshape(-1)
    p1d = events_list[:, :, 3].reshape(-1)
    w1p = jnp.zeros((_HP, 1), jnp.float32).at[:100, 0].set(W1[:, 0])
    cs = (jnp.arange(_C, dtype=jnp.float32) / (_C - 1))[None, :]
    b1s = (jnp.zeros((_HP, _C), jnp.float32)
           .at[:100, :].set(b1[:, None] - W1[:, 0:1] * cs))
    npair = (_C + 1) // 2
    pa = jnp.minimum(jnp.arange(npair) * 2, _C - 1)
    pb = jnp.minimum(jnp.arange(npair) * 2 + 1, _C - 1)
    b1p = jnp.concatenate([b1s[:, pa], b1s[:, pb]], axis=0)
    w2s = jnp.zeros((_HP, _HP), jnp.float32).at[:100, :100].set(W2)
    zz = jnp.zeros((_HP, _HP), jnp.float32)
    w2p = jnp.concatenate(
        [jnp.concatenate([w2s, zz], axis=1),
         jnp.concatenate([zz, w2s], axis=1)], axis=0)
    b2s = jnp.zeros((_HP, 1), jnp.float32).at[:100, 0].set(b2)
    b2p = jnp.concatenate([b2s, b2s], axis=0)
    w3p = jnp.zeros((_HP, 1), jnp.float32).at[:100, 0].set(W3[0, :])
    b3p = b3.reshape(1, 1)

    vals = _tc_values(t1d, w1p, b1p, w2p, b2p, w3p, b3p)
    vox = _sc_scatter(x1d, y1d, p1d, vals, num_voxels, half)
    return vox.reshape(Bn, 2 * _C, _H, _W)


# speedup vs baseline: 1.0352x; 1.0352x over previous
import jax
import jax.numpy as jnp
from jax import lax
from jax.experimental import pallas as pl
from jax.experimental.pallas import tpu as pltpu
from jax.experimental.pallas import tpu_sc as plsc

_C, _H, _W = 9, 180, 240
_WH = _W * _H
_HP = 128
_EV_CHUNK = 16384

_EC = 5000
_ZB = 6480


def _tc_body(t_ref, w1_ref, b1s_ref, w2_ref, b2_ref, w3_ref, b3_ref,
             *o_refs):
    t = t_ref[...][None, :]
    w1 = w1_ref[...]
    w2 = w2_ref[...]
    b2 = b2_ref[...]
    w3 = w3_ref[...]
    b3 = b3_ref[0, 0]
    g = w1 * t
    for i in range(_C):
        h1 = g + b1s_ref[:, i:i + 1]
        h1 = jnp.maximum(h1, 0.1 * h1)
        h2 = jnp.dot(w2, h1, preferred_element_type=jnp.float32) + b2
        h2 = jnp.maximum(h2, 0.1 * h2)
        o = jnp.sum(h2 * w3, axis=0, keepdims=True) + b3
        o_refs[i][...] = (t * o)[0]


def _tc_values(t1d, w1p, b1p, w2p, b2p, w3p, b3p):
    nev = t1d.shape[0]
    chunk = _EV_CHUNK
    nblk = pl.cdiv(nev, chunk)
    return pl.pallas_call(
        _tc_body,
        grid=(nblk,),
        in_specs=[
            pl.BlockSpec((chunk,), lambda i: (i,)),
            pl.BlockSpec((_HP, 1), lambda i: (0, 0)),
            pl.BlockSpec((_HP, _C), lambda i: (0, 0)),
            pl.BlockSpec((_HP, _HP), lambda i: (0, 0)),
            pl.BlockSpec((_HP, 1), lambda i: (0, 0)),
            pl.BlockSpec((_HP, 1), lambda i: (0, 0)),
            pl.BlockSpec((1, 1), lambda i: (0, 0)),
        ],
        out_specs=[pl.BlockSpec((chunk,), lambda i: (i,))
                   for _ in range(_C)],
        out_shape=[jax.ShapeDtypeStruct((nev,), jnp.float32)
                   for _ in range(_C)],
    )(t1d, w1p, b1p, w2p, b2p, w3p, b3p)


def _sc_scatter(x1d, y1d, p1d, vals, num_voxels, half):
    nev = x1d.shape[0]
    ev_per_core = nev // 2
    ev_per_tile = ev_per_core // 16
    n_chunks = ev_per_tile // _EC
    nfull = _EC // 16
    has_tail = (_EC % 16) != 0
    gwords = half // 16
    mesh = plsc.VectorSubcoreMesh(core_axis_name="c", subcore_axis_name="s")

    def body(x_hbm, y_hbm, p_hbm, *rest):
        v_hbms = rest[:_C]
        out_hbm = rest[_C]
        (x_v, y_v, p_v, idx_v, val_v, zb_v, grid_s) = rest[_C + 1:]
        cid = lax.axis_index("c")
        sid = lax.axis_index("s")

        def zf(i, carry):
            zb_v[pl.ds(i * 16, 16)] = jnp.zeros((16,), jnp.float32)
            return carry
        lax.fori_loop(0, _ZB // 16, zf, 0)
        gbase = sid * gwords

        def zc(j, carry):
            pltpu.sync_copy(zb_v, grid_s.at[pl.ds(gbase + j * _ZB, _ZB)])
            return carry
        lax.fori_loop(0, gwords // _ZB, zc, 0)
        plsc.subcore_barrier()

        ebase = cid * ev_per_core + sid * ev_per_tile
        batch = cid * 2 + sid // 8
        bterm = batch.astype(jnp.float32) * 777600.0
        loc = cid * half

        def base_of(s):
            xx = x_v[pl.ds(s, 16)]
            yy = y_v[pl.ds(s, 16)]
            pp = p_v[pl.ds(s, 16)]
            p2 = (pp + 1.0) * 0.5
            b = xx + 240.0 * yy
            b = b + 388800.0 * p2
            return b + bterm

        def to_idx(f, off):
            f = jnp.clip(f + off, 0.0, float(num_voxels - 1))
            return f.astype(jnp.int32) - loc

        def sc_chunk(r, carry):
            e0 = ebase + r * _EC
            pltpu.sync_copy(x_hbm.at[pl.ds(e0, _EC)], x_v)
            pltpu.sync_copy(y_hbm.at[pl.ds(e0, _EC)], y_v)
            pltpu.sync_copy(p_hbm.at[pl.ds(e0, _EC)], p_v)

            if has_tail:
                zb_v[pl.ds(0, 16)] = base_of(_EC - 16)

            def bvec(v, carry2):
                s = v * 16
                x_v[pl.ds(s, 16)] = base_of(s)
                return carry2
            lax.fori_loop(0, nfull, bvec, 0)

            for i in range(_C):
                off = float(_WH * i)

                def ivec(v, carry3):
                    s = v * 16
                    idx_v[pl.ds(s, 16)] = to_idx(x_v[pl.ds(s, 16)], off)
                    return carry3
                lax.fori_loop(0, nfull, ivec, 0)
                if has_tail:
                    idx_v[pl.ds(_EC - 16, 16)] = to_idx(zb_v[pl.ds(0, 16)],
                                                        off)
                pltpu.sync_copy(v_hbms[i].at[pl.ds(e0, _EC)], val_v)
                pltpu.sync_copy(val_v, grid_s.at[idx_v], add=True)
            return carry
        lax.fori_loop(0, n_chunks, sc_chunk, 0)
        plsc.subcore_barrier()

        def oc(j, carry):
            pltpu.sync_copy(grid_s.at[pl.ds(gbase + j * _ZB, _ZB)], zb_v)
            pltpu.sync_copy(zb_v,
                            out_hbm.at[pl.ds(cid * half + gbase + j * _ZB,
                                             _ZB)])
            return carry
        lax.fori_loop(0, gwords // _ZB, oc, 0)

    f = pl.kernel(
        body,
        out_type=jax.ShapeDtypeStruct((num_voxels,), jnp.float32),
        mesh=mesh,
        scratch_types=[
            pltpu.VMEM((_EC,), jnp.float32),
            pltpu.VMEM((_EC,), jnp.float32),
            pltpu.VMEM((_EC,), jnp.float32),
            pltpu.VMEM((_EC,), jnp.int32),
            pltpu.VMEM((_EC,), jnp.float32),
            pltpu.VMEM((_ZB,), jnp.float32),
            pltpu.VMEM_SHARED((half,), jnp.float32),
        ],
    )
    return f(x1d, y1d, p1d, *vals)


def kernel(events_list, W1, b1, W2, b2, W3, b3):
    Bn, Nn = events_list.shape[0], events_list.shape[1]
    num_voxels = 2 * _C * _H * _W * Bn
    half = num_voxels // 2

    x1d = events_list[:, :, 0].reshape(-1)
    y1d = events_list[:, :, 1].reshape(-1)
    t1d = events_list[:, :, 2].reshape(-1)
    p1d = events_list[:, :, 3].reshape(-1)
    w1p = jnp.zeros((_HP, 1), jnp.float32).at[:100, 0].set(W1[:, 0])
    cs = (jnp.arange(_C, dtype=jnp.float32) / (_C - 1))[None, :]
    b1s = (jnp.zeros((_HP, _C), jnp.float32)
           .at[:100, :].set(b1[:, None] - W1[:, 0:1] * cs))
    b1p = b1s
    w2p = jnp.zeros((_HP, _HP), jnp.float32).at[:100, :100].set(W2)
    b2p = jnp.zeros((_HP, 1), jnp.float32).at[:100, 0].set(b2)
    w3p = jnp.zeros((_HP, 1), jnp.float32).at[:100, 0].set(W3[0, :])
    b3p = b3.reshape(1, 1)

    vals = _tc_values(t1d, w1p, b1p, w2p, b2p, w3p, b3p)
    vox = _sc_scatter(x1d, y1d, p1d, vals, num_voxels, half)
    return vox.reshape(Bn, 2 * _C, _H, _W)
